# baseline (speedup 1.0000x reference)
import functools

import jax
import jax.numpy as jnp
from jax.experimental import pallas as pl

_H = 1000
_W = 2000
_PS = 20
_PCOLS = _W // _PS
_NT = 5
_R = 5
_GRID = _H // (_PS * _R)


def _strip_kernel(x_ref, y_ref, w_ref, o_ref):
    lane = jax.lax.broadcasted_iota(jnp.int32, (_PCOLS, _W), 1)
    sub = jax.lax.broadcasted_iota(jnp.int32, (_PCOLS, _W), 0)
    e = (lane // _PS == sub).astype(jnp.float32)
    xall = x_ref[0, 0]
    yall = y_ref[0, 0]
    for r in range(_R):
        w = w_ref[0, r]
        ew = jax.lax.dot_general(
            w, e, (((1,), (0,)), ((), ())),
            preferred_element_type=jnp.float32)

        def row(k, ew=ew):
            return jax.lax.slice_in_dim(ew, k, k + 1, axis=0)

        x = jax.lax.slice_in_dim(xall, r * _PS, (r + 1) * _PS, axis=0)
        y = jax.lax.slice_in_dim(yall, r * _PS, (r + 1) * _PS, axis=0)
        for c in range(3):
            base = c * 11
            px = row(base + 4)
            for t in (3, 2, 1, 0):
                px = px * x + row(base + t)
            py = row(base + 9)
            for t in (8, 7, 6, 5):
                py = py * y + row(base + t)
            o_ref[c, 0, pl.ds(r * _PS, _PS)] = px + py + row(base + 10)


@functools.partial(jax.jit, static_argnums=())
def kernel(pix_coord, coefficients, bias):
    xyt = pix_coord.T.reshape(2, _GRID, _R * _PS, _W)
    w = jnp.concatenate(
        [coefficients[..., 0], coefficients[..., 1], bias[..., None]],
        axis=-1)
    w = w.reshape(_H // _PS, _PCOLS, 33).transpose(0, 2, 1)
    w = w.reshape(_GRID, _R, 33, _PCOLS)

    out = pl.pallas_call(
        _strip_kernel,
        grid=(_GRID,),
        in_specs=[
            pl.BlockSpec((1, 1, _R * _PS, _W), lambda i: (0, i, 0, 0)),
            pl.BlockSpec((1, 1, _R * _PS, _W), lambda i: (1, i, 0, 0)),
            pl.BlockSpec((1, _R, 33, _PCOLS), lambda i: (i, 0, 0, 0)),
        ],
        out_specs=pl.BlockSpec((3, 1, _R * _PS, _W), lambda i: (0, i, 0, 0)),
        out_shape=jax.ShapeDtypeStruct((3, _GRID, _R * _PS, _W), jnp.float32),
    )(xyt, xyt, w)
    return out.reshape(3, _H * _W)


# speedup vs baseline: 1.0003x; 1.0003x over previous
import functools

import jax
import jax.numpy as jnp
from jax.experimental import pallas as pl

_H = 1000
_W = 2000
_PS = 20
_PCOLS = _W // _PS
_NT = 5
_R = 5
_GRID = _H // (_PS * _R)


def _strip_kernel(xy_ref, w_ref, o_ref):
    lane = jax.lax.broadcasted_iota(jnp.int32, (_PCOLS, _W), 1)
    sub = jax.lax.broadcasted_iota(jnp.int32, (_PCOLS, _W), 0)
    e = (lane // _PS == sub).astype(jnp.float32)
    xall = xy_ref[0, 0]
    yall = xy_ref[1, 0]
    for r in range(_R):
        w = w_ref[0, r]
        ew = jax.lax.dot_general(
            w, e, (((1,), (0,)), ((), ())),
            preferred_element_type=jnp.float32)

        def row(k, ew=ew):
            return jax.lax.slice_in_dim(ew, k, k + 1, axis=0)

        x = jax.lax.slice_in_dim(xall, r * _PS, (r + 1) * _PS, axis=0)
        y = jax.lax.slice_in_dim(yall, r * _PS, (r + 1) * _PS, axis=0)
        for c in range(3):
            base = c * 11
            px = row(base + 4)
            for t in (3, 2, 1, 0):
                px = px * x + row(base + t)
            py = row(base + 9)
            for t in (8, 7, 6, 5):
                py = py * y + row(base + t)
            o_ref[c, 0, pl.ds(r * _PS, _PS)] = px + py + row(base + 10)


@functools.partial(jax.jit, static_argnums=())
def kernel(pix_coord, coefficients, bias):
    xyt = pix_coord.T.reshape(2, _GRID, _R * _PS, _W)
    w = jnp.concatenate(
        [coefficients[..., 0], coefficients[..., 1], bias[..., None]],
        axis=-1)
    w = w.reshape(_H // _PS, _PCOLS, 33).transpose(0, 2, 1)
    w = w.reshape(_GRID, _R, 33, _PCOLS)

    out = pl.pallas_call(
        _strip_kernel,
        grid=(_GRID,),
        in_specs=[
            pl.BlockSpec((2, 1, _R * _PS, _W), lambda i: (0, i, 0, 0)),
            pl.BlockSpec((1, _R, 33, _PCOLS), lambda i: (i, 0, 0, 0)),
        ],
        out_specs=pl.BlockSpec((3, 1, _R * _PS, _W), lambda i: (0, i, 0, 0)),
        out_shape=jax.ShapeDtypeStruct((3, _GRID, _R * _PS, _W), jnp.float32),
    )(xyt, w)
    return out.reshape(3, _H * _W)


# speedup vs baseline: 3.5661x; 3.5652x over previous
import functools

import jax
import jax.numpy as jnp
from jax.experimental import pallas as pl

_H = 1000
_W = 2000
_PS = 20
_PCOLS = _W // _PS
_LW = 16000
_SR = 5
_GRID = (_H * _W) // (_LW * _SR)


def _strip_kernel(xy_ref, w_ref, e_ref, o_ref):
    x = xy_ref[0, 0]
    y = xy_ref[1, 0]
    e16 = e_ref[...]
    wa = w_ref[0, 0]
    wb = w_ref[0, 1]
    sub = jax.lax.broadcasted_iota(jnp.int32, (_SR, _LW), 0)
    lane = jax.lax.broadcasted_iota(jnp.int32, (_SR, _LW), 1)
    amask = 8 * sub + lane // _W < _PS

    for c in range(3):
        ewa = jax.lax.dot_general(
            jax.lax.slice_in_dim(wa, c * 11, (c + 1) * 11, axis=0), e16,
            (((1,), (0,)), ((), ())), preferred_element_type=jnp.float32)
        ewb = jax.lax.dot_general(
            jax.lax.slice_in_dim(wb, c * 11, (c + 1) * 11, axis=0), e16,
            (((1,), (0,)), ((), ())), preferred_element_type=jnp.float32)

        def row(k, ewa=ewa, ewb=ewb):
            ra = jax.lax.slice_in_dim(ewa, k, k + 1, axis=0)
            rb = jax.lax.slice_in_dim(ewb, k, k + 1, axis=0)
            return jnp.where(amask, ra, rb)

        px = row(4)
        for t in (3, 2, 1, 0):
            px = px * x + row(t)
        py = row(9)
        for t in (8, 7, 6, 5):
            py = py * y + row(t)
        o_ref[c, 0] = px + py + row(10)


@functools.partial(jax.jit, static_argnums=())
def kernel(pix_coord, coefficients, bias):
    xyt = pix_coord.T.reshape(2, _GRID, _SR, _LW)
    w = jnp.concatenate(
        [coefficients[..., 0], coefficients[..., 1], bias[..., None]],
        axis=-1)
    w = w.reshape(_H // _PS, _PCOLS, 33).transpose(0, 2, 1)
    w = w.reshape(_GRID, 2, 33, _PCOLS)
    e16 = ((jnp.arange(_LW, dtype=jnp.int32)[None, :] % _W) // _PS ==
           jnp.arange(_PCOLS, dtype=jnp.int32)[:, None]).astype(jnp.float32)

    out = pl.pallas_call(
        _strip_kernel,
        grid=(_GRID,),
        in_specs=[
            pl.BlockSpec((2, 1, _SR, _LW), lambda i: (0, i, 0, 0)),
            pl.BlockSpec((1, 2, 33, _PCOLS), lambda i: (i, 0, 0, 0)),
            pl.BlockSpec((_PCOLS, _LW), lambda i: (0, 0)),
        ],
        out_specs=pl.BlockSpec((3, 1, _SR, _LW), lambda i: (0, i, 0, 0)),
        out_shape=jax.ShapeDtypeStruct((3, _GRID, _SR, _LW), jnp.float32),
    )(xyt, w, e16)
    return out.reshape(3, _H * _W)


# speedup vs baseline: 4.0733x; 1.1422x over previous
import functools

import jax
import jax.numpy as jnp
from jax.experimental import pallas as pl

_H = 1000
_W = 2000
_PS = 20
_PCOLS = _W // _PS
_LW = 16000
_SR = 5
_GRID = (_H * _W) // (_LW * _SR)


def _strip_kernel(xy_ref, w_ref, e_ref, o_ref):
    x = xy_ref[0, 0]
    y = xy_ref[1, 0]
    e16 = e_ref[...]
    wa = w_ref[0, 0]
    wb = w_ref[0, 1]
    sub = jax.lax.broadcasted_iota(jnp.int32, (_SR, _LW), 0)
    lane = jax.lax.broadcasted_iota(jnp.int32, (_SR, _LW), 1)
    amask = 8 * sub + lane // _W < _PS

    ewa_all = jax.lax.dot_general(
        wa, e16, (((1,), (0,)), ((), ())), preferred_element_type=jnp.float32)
    ewb_all = jax.lax.dot_general(
        wb, e16, (((1,), (0,)), ((), ())), preferred_element_type=jnp.float32)

    for c in range(3):
        def row(k, c=c):
            ra = jax.lax.slice_in_dim(ewa_all, c * 11 + k, c * 11 + k + 1, axis=0)
            rb = jax.lax.slice_in_dim(ewb_all, c * 11 + k, c * 11 + k + 1, axis=0)
            return jnp.where(amask, ra, rb)

        px = row(4)
        for t in (3, 2, 1, 0):
            px = px * x + row(t)
        py = row(9)
        for t in (8, 7, 6, 5):
            py = py * y + row(t)
        o_ref[c, 0] = px + py + row(10)


@functools.partial(jax.jit, static_argnums=())
def kernel(pix_coord, coefficients, bias):
    xyt = pix_coord.T.reshape(2, _GRID, _SR, _LW)
    w = jnp.concatenate(
        [coefficients[..., 0], coefficients[..., 1], bias[..., None]],
        axis=-1)
    w = w.reshape(_H // _PS, _PCOLS, 33).transpose(0, 2, 1)
    w = w.reshape(_GRID, 2, 33, _PCOLS)
    e16 = ((jnp.arange(_LW, dtype=jnp.int32)[None, :] % _W) // _PS ==
           jnp.arange(_PCOLS, dtype=jnp.int32)[:, None]).astype(jnp.float32)

    out = pl.pallas_call(
        _strip_kernel,
        grid=(_GRID,),
        in_specs=[
            pl.BlockSpec((2, 1, _SR, _LW), lambda i: (0, i, 0, 0)),
            pl.BlockSpec((1, 2, 33, _PCOLS), lambda i: (i, 0, 0, 0)),
            pl.BlockSpec((_PCOLS, _LW), lambda i: (0, 0)),
        ],
        out_specs=pl.BlockSpec((3, 1, _SR, _LW), lambda i: (0, i, 0, 0)),
        out_shape=jax.ShapeDtypeStruct((3, _GRID, _SR, _LW), jnp.float32),
    )(xyt, w, e16)
    return out.reshape(3, _H * _W)


# speedup vs baseline: 4.0850x; 1.0029x over previous
import functools

import jax
import jax.numpy as jnp
from jax.experimental import pallas as pl

_H = 1000
_W = 2000
_PS = 20
_PCOLS = _W // _PS
_LW = 16000
_SR = 5
_GRID = (_H * _W) // (_LW * _SR)


def _strip_kernel(xy_ref, w_ref, e_ref, o_ref):
    x = xy_ref[0, 0]
    y = xy_ref[1, 0]
    e16 = e_ref[...]
    wa = w_ref[0, 0]
    wb = w_ref[0, 1]
    sub = jax.lax.broadcasted_iota(jnp.int32, (_SR, _LW), 0)
    lane = jax.lax.broadcasted_iota(jnp.int32, (_SR, _LW), 1)
    amask = 8 * sub + lane // _W < _PS

    ewa_all = jax.lax.dot_general(
        wa.astype(jnp.bfloat16), e16, (((1,), (0,)), ((), ())),
        preferred_element_type=jnp.float32)
    ewb_all = jax.lax.dot_general(
        wb.astype(jnp.bfloat16), e16, (((1,), (0,)), ((), ())),
        preferred_element_type=jnp.float32)

    for c in range(3):
        def row(k, c=c):
            ra = jax.lax.slice_in_dim(ewa_all, c * 11 + k, c * 11 + k + 1, axis=0)
            rb = jax.lax.slice_in_dim(ewb_all, c * 11 + k, c * 11 + k + 1, axis=0)
            return jnp.where(amask, ra, rb)

        px = row(4)
        for t in (3, 2, 1, 0):
            px = px * x + row(t)
        py = row(9)
        for t in (8, 7, 6, 5):
            py = py * y + row(t)
        o_ref[c, 0] = px + py + row(10)


@functools.partial(jax.jit, static_argnums=())
def kernel(pix_coord, coefficients, bias):
    xyt = pix_coord.T.reshape(2, _GRID, _SR, _LW)
    w = jnp.concatenate(
        [coefficients[..., 0], coefficients[..., 1], bias[..., None]],
        axis=-1)
    w = w.reshape(_H // _PS, _PCOLS, 33).transpose(0, 2, 1)
    w = w.reshape(_GRID, 2, 33, _PCOLS)
    e16 = ((jnp.arange(_LW, dtype=jnp.int32)[None, :] % _W) // _PS ==
           jnp.arange(_PCOLS, dtype=jnp.int32)[:, None]).astype(jnp.bfloat16)

    out = pl.pallas_call(
        _strip_kernel,
        grid=(_GRID,),
        in_specs=[
            pl.BlockSpec((2, 1, _SR, _LW), lambda i: (0, i, 0, 0)),
            pl.BlockSpec((1, 2, 33, _PCOLS), lambda i: (i, 0, 0, 0)),
            pl.BlockSpec((_PCOLS, _LW), lambda i: (0, 0)),
        ],
        out_specs=pl.BlockSpec((3, 1, _SR, _LW), lambda i: (0, i, 0, 0)),
        out_shape=jax.ShapeDtypeStruct((3, _GRID, _SR, _LW), jnp.float32),
    )(xyt, w, e16)
    return out.reshape(3, _H * _W)
